# baseline (speedup 1.0000x reference)
import functools

import jax
import jax.numpy as jnp
from jax import lax
from jax.experimental import pallas as pl
from jax.experimental.pallas import tpu as pltpu
from jax.experimental.pallas import tpu_sc as plsc

B = 4096
L = 50
D = 64
NW = 32
BW = B // NW
CH = 64
NCH = (BW * L) // CH
NROWP = 56
MAGIC = 41944


def _body(up2, ip2, uid_hbm, iid_hbm, phys_hbm, parf_hbm, uparf_hbm,
          iparf_hbm, histn_hbm, out_hbm,
          phys_v, rows_v, parf_v, acc_v, urows_v, irows_v, histn_v,
          uparf_v, iparf_v, uidx_v, iidx_v, uphys_v, iphys_v, out_buf,
          sem_g0, sem_g1, sem_u, sem_i, sem_n):
    c = lax.axis_index("c")
    s = lax.axis_index("s")
    wid = s * 2 + c
    base = wid * BW

    pltpu.sync_copy(phys_hbm.at[wid], phys_v)

    def gather_chunk(cc, p, sem):
        r = lax.shift_right_logical(cc, 1)
        co = jnp.bitwise_and(cc, 1) * CH
        pltpu.make_async_copy(
            ip2.at[phys_v.at[r, pl.ds(co, CH)]], rows_v.at[p], sem).start()
        pltpu.make_async_copy(
            parf_hbm.at[wid, pl.ds(cc * 8, 8), :], parf_v.at[p], sem).start()

    def wait_chunk(cc, p, sem):
        r = lax.shift_right_logical(cc, 1)
        co = jnp.bitwise_and(cc, 1) * CH
        pltpu.make_async_copy(
            ip2.at[phys_v.at[r, pl.ds(co, CH)]], rows_v.at[p], sem).wait()
        pltpu.make_async_copy(
            parf_hbm.at[wid, pl.ds(cc * 8, 8), :], parf_v.at[p], sem).wait()

    gather_chunk(jnp.int32(0), 0, sem_g0)
    gather_chunk(jnp.int32(1), 1, sem_g1)

    pltpu.make_async_copy(uid_hbm.at[pl.ds(base, BW)], uidx_v, sem_u).start()
    pltpu.make_async_copy(iid_hbm.at[pl.ds(base, BW)], iidx_v, sem_i).start()
    pltpu.make_async_copy(histn_hbm.at[wid], histn_v, sem_n).start()
    pltpu.make_async_copy(uparf_hbm.at[wid], uparf_v, sem_n).start()
    pltpu.make_async_copy(iparf_hbm.at[wid], iparf_v, sem_n).start()

    pltpu.make_async_copy(uid_hbm.at[pl.ds(base, BW)], uidx_v, sem_u).wait()
    for k in range(BW // 16):
        sl = pl.ds(k * 16, 16)
        uphys_v[sl] = lax.shift_right_logical(uidx_v[sl], 1)
    pltpu.make_async_copy(up2.at[uphys_v], urows_v, sem_u).start()
    pltpu.make_async_copy(iid_hbm.at[pl.ds(base, BW)], iidx_v, sem_i).wait()
    for k in range(BW // 16):
        sl = pl.ds(k * 16, 16)
        iphys_v[sl] = lax.shift_right_logical(iidx_v[sl], 1)
    pltpu.make_async_copy(ip2.at[iphys_v], irows_v, sem_i).start()

    zero = jnp.zeros((16,), jnp.float32)

    def zero_body(i, carry):
        for j in range(4):
            acc_v[i, pl.ds(j * 16, 16)] = zero
        return carry

    lax.fori_loop(0, BW, zero_body, 0)

    def main_body(it, carry):
        for p in range(2):
            cc = it * 2 + p
            sem = sem_g0 if p == 0 else sem_g1
            wait_chunk(cc, p, sem)
            for i in range(CH):
                dst = lax.shift_right_logical((cc * CH + i) * MAGIC, 21)
                pf = parf_v[p, i // 8, pl.ds((i % 8) * 16, 16)]
                for j in range(4):
                    lo = rows_v[p, i, pl.ds(j * 16, 16)]
                    hi = rows_v[p, i, pl.ds(D + j * 16, 16)]
                    v = lo + pf * (hi - lo)
                    plsc.addupdate(acc_v.at[dst, pl.ds(j * 16, 16)], v)

            @pl.when(cc + 2 < NCH)
            def _():
                gather_chunk(cc + 2, p, sem)
        return carry

    lax.fori_loop(0, NCH // 2, main_body, 0)

    pltpu.make_async_copy(histn_hbm.at[wid], histn_v, sem_n).wait()
    pltpu.make_async_copy(uparf_hbm.at[wid], uparf_v, sem_n).wait()
    pltpu.make_async_copy(iparf_hbm.at[wid], iparf_v, sem_n).wait()
    pltpu.make_async_copy(up2.at[uphys_v], urows_v, sem_u).wait()
    pltpu.make_async_copy(ip2.at[iphys_v], irows_v, sem_i).wait()

    def asm_body(b, carry):
        r2 = lax.shift_right_logical(b, 1)
        co = jnp.bitwise_and(b, 1) * 64
        cnt = jnp.zeros((16,), jnp.int32)
        for k in range(4):
            x = histn_v[r2, pl.ds(co + k * 16, 16)]
            cnt = cnt + plsc.all_reduce_population_count(x != 0)
        rec = 1.0 / (cnt.astype(jnp.float32) + 1e-9)
        r3 = lax.shift_right_logical(b, 3)
        co3 = jnp.bitwise_and(b, 7) * 16
        upf = uparf_v[r3, pl.ds(co3, 16)]
        ipf = iparf_v[r3, pl.ds(co3, 16)]
        for j in range(4):
            sl = pl.ds(j * 16, 16)
            ulo = urows_v[b, sl]
            uhi = urows_v[b, pl.ds(D + j * 16, 16)]
            out_buf[b, sl] = ulo + upf * (uhi - ulo)
            ilo = irows_v[b, sl]
            ihi = irows_v[b, pl.ds(D + j * 16, 16)]
            out_buf[b, pl.ds(D + j * 16, 16)] = ilo + ipf * (ihi - ilo)
            out_buf[b, pl.ds(2 * D + j * 16, 16)] = acc_v[b, sl] * rec
            out_buf[b, pl.ds(3 * D + j * 16, 16)] = zero
        return carry

    lax.fori_loop(0, BW, asm_body, 0)
    pltpu.sync_copy(out_buf, out_hbm.at[pl.ds(base, BW), :])


_sc_call = functools.partial(
    pl.kernel,
    mesh=plsc.VectorSubcoreMesh(core_axis_name="c", subcore_axis_name="s"),
    out_type=jax.ShapeDtypeStruct((B, 4 * D), jnp.float32),
    compiler_params=pltpu.CompilerParams(needs_layout_passes=False),
    scratch_types=[
        pltpu.VMEM((NROWP, 128), jnp.int32),
        pltpu.VMEM((2, CH, 128), jnp.float32),
        pltpu.VMEM((2, 8, 128), jnp.float32),
        pltpu.VMEM((BW, D), jnp.float32),
        pltpu.VMEM((BW, 128), jnp.float32),
        pltpu.VMEM((BW, 128), jnp.float32),
        pltpu.VMEM((D, 128), jnp.int32),
        pltpu.VMEM((16, 128), jnp.float32),
        pltpu.VMEM((16, 128), jnp.float32),
        pltpu.VMEM((BW,), jnp.int32),
        pltpu.VMEM((BW,), jnp.int32),
        pltpu.VMEM((BW,), jnp.int32),
        pltpu.VMEM((BW,), jnp.int32),
        pltpu.VMEM((BW, 4 * D), jnp.float32),
        pltpu.SemaphoreType.DMA,
        pltpu.SemaphoreType.DMA,
        pltpu.SemaphoreType.DMA,
        pltpu.SemaphoreType.DMA,
        pltpu.SemaphoreType.DMA,
    ],
)(_body)


@jax.jit
def kernel(user_tab, item_tab, user_id, item_id, history_item_id):
    uid = user_id.astype(jnp.int32)
    iid = item_id.astype(jnp.int32)
    hid = history_item_id.astype(jnp.int32)

    up2 = user_tab.reshape(-1, 128)
    ip2 = item_tab.reshape(-1, 128)

    phys = jnp.right_shift(hid, 1).reshape(NW, BW * L)
    phys = jnp.pad(phys, ((0, 0), (0, NROWP * 128 - BW * L)))
    phys = phys.reshape(NW, NROWP, 128)

    ones16 = jnp.ones((16,), jnp.float32)
    parf = (jnp.bitwise_and(hid, 1).astype(jnp.float32)
            .reshape(NW, BW * L, 1) * ones16).reshape(NW, NCH * 8, 128)
    uparf = (jnp.bitwise_and(uid, 1).astype(jnp.float32)
             .reshape(NW, BW, 1) * ones16).reshape(NW, 16, 128)
    iparf = (jnp.bitwise_and(iid, 1).astype(jnp.float32)
             .reshape(NW, BW, 1) * ones16).reshape(NW, 16, 128)

    histn = jnp.pad(hid, ((0, 0), (0, D - L))).reshape(NW, D, 128)

    out = _sc_call(up2, ip2, uid, iid, phys, parf, uparf, iparf, histn)
    return out[:, :3 * D]


# speedup vs baseline: 1.3492x; 1.3492x over previous
import functools

import jax
import jax.numpy as jnp
from jax import lax
from jax.experimental import pallas as pl
from jax.experimental.pallas import tpu as pltpu
from jax.experimental.pallas import tpu_sc as plsc

B = 4096
L = 50
D = 64
NW = 32
BW = B // NW
CH = 64
NCH = (BW * L) // CH
NBUF = 4
NROWP = 56
MAGIC = 41944


def _body(up2, ip2, uid_hbm, iid_hbm, phys_hbm, histn_hbm, out_hbm,
          phys_v, rows_v, urows_v, irows_v, histn_v,
          uidx_v, iidx_v, out_buf,
          sem_g0, sem_g1, sem_g2, sem_g3, sem_u, sem_i, sem_n):
    c = lax.axis_index("c")
    s = lax.axis_index("s")
    wid = s * 2 + c
    base = wid * BW
    sems = [sem_g0, sem_g1, sem_g2, sem_g3]

    pltpu.sync_copy(phys_hbm.at[wid], phys_v)

    def gather_chunk(cc, p):
        r = lax.shift_right_logical(cc, 1)
        co = jnp.bitwise_and(cc, 1) * CH
        pltpu.make_async_copy(
            ip2.at[phys_v.at[r, pl.ds(co, CH)]], rows_v.at[p], sems[p]).start()

    def wait_chunk(cc, p):
        r = lax.shift_right_logical(cc, 1)
        co = jnp.bitwise_and(cc, 1) * CH
        pltpu.make_async_copy(
            ip2.at[phys_v.at[r, pl.ds(co, CH)]], rows_v.at[p], sems[p]).wait()

    for p in range(NBUF):
        gather_chunk(jnp.int32(p), p)

    pltpu.make_async_copy(uid_hbm.at[pl.ds(base, BW)], uidx_v, sem_u).start()
    pltpu.make_async_copy(iid_hbm.at[pl.ds(base, BW)], iidx_v, sem_i).start()
    pltpu.make_async_copy(histn_hbm.at[wid], histn_v, sem_n).start()

    pltpu.make_async_copy(uid_hbm.at[pl.ds(base, BW)], uidx_v, sem_u).wait()
    pltpu.make_async_copy(up2.at[uidx_v], urows_v, sem_u).start()
    pltpu.make_async_copy(iid_hbm.at[pl.ds(base, BW)], iidx_v, sem_i).wait()
    pltpu.make_async_copy(ip2.at[iidx_v], irows_v, sem_i).start()

    zero = jnp.zeros((16,), jnp.float32)

    def zero_body(i, carry):
        for j in range(8):
            out_buf[i, pl.ds(128 + j * 16, 16)] = zero
        return carry

    lax.fori_loop(0, BW, zero_body, 0)

    def main_body(it, carry):
        for p in range(NBUF):
            cc = it * NBUF + p
            wait_chunk(cc, p)
            for i in range(CH):
                dst = lax.shift_right_logical((cc * CH + i) * MAGIC, 21)
                for j in range(4):
                    v = rows_v[p, i, pl.ds(j * 16, 16)]
                    plsc.addupdate(
                        out_buf.at[dst, pl.ds(2 * D + j * 16, 16)], v)

            @pl.when(cc + NBUF < NCH)
            def _():
                gather_chunk(cc + NBUF, p)
        return carry

    lax.fori_loop(0, NCH // NBUF, main_body, 0)

    pltpu.make_async_copy(histn_hbm.at[wid], histn_v, sem_n).wait()
    pltpu.make_async_copy(up2.at[uidx_v], urows_v, sem_u).wait()
    pltpu.make_async_copy(ip2.at[iidx_v], irows_v, sem_i).wait()

    def asm_body(b, carry):
        r2 = lax.shift_right_logical(b, 1)
        co = jnp.bitwise_and(b, 1) * 64
        cnt = jnp.zeros((16,), jnp.int32)
        for k in range(4):
            x = histn_v[r2, pl.ds(co + k * 16, 16)]
            cnt = cnt + plsc.all_reduce_population_count(x != 0)
        rec = 1.0 / (cnt.astype(jnp.float32) + 1e-9)
        for j in range(4):
            sl = pl.ds(j * 16, 16)
            out_buf[b, sl] = urows_v[b, sl]
            out_buf[b, pl.ds(D + j * 16, 16)] = irows_v[b, sl]
            hsl = pl.ds(2 * D + j * 16, 16)
            out_buf[b, hsl] = out_buf[b, hsl] * rec
        return carry

    lax.fori_loop(0, BW, asm_body, 0)
    pltpu.sync_copy(out_buf, out_hbm.at[pl.ds(base, BW), :])


_sc_call = functools.partial(
    pl.kernel,
    mesh=plsc.VectorSubcoreMesh(core_axis_name="c", subcore_axis_name="s"),
    out_type=jax.ShapeDtypeStruct((B, 4 * D), jnp.float32),
    compiler_params=pltpu.CompilerParams(needs_layout_passes=False),
    scratch_types=[
        pltpu.VMEM((NROWP, 128), jnp.int32),
        pltpu.VMEM((NBUF, CH, 128), jnp.float32),
        pltpu.VMEM((BW, 128), jnp.float32),
        pltpu.VMEM((BW, 128), jnp.float32),
        pltpu.VMEM((D, 128), jnp.int32),
        pltpu.VMEM((BW,), jnp.int32),
        pltpu.VMEM((BW,), jnp.int32),
        pltpu.VMEM((BW, 4 * D), jnp.float32),
        pltpu.SemaphoreType.DMA,
        pltpu.SemaphoreType.DMA,
        pltpu.SemaphoreType.DMA,
        pltpu.SemaphoreType.DMA,
        pltpu.SemaphoreType.DMA,
        pltpu.SemaphoreType.DMA,
        pltpu.SemaphoreType.DMA,
    ],
)(_body)


@jax.jit
def kernel(user_tab, item_tab, user_id, item_id, history_item_id):
    uid = user_id.astype(jnp.int32)
    iid = item_id.astype(jnp.int32)
    hid = history_item_id.astype(jnp.int32)

    up2 = jnp.pad(user_tab, ((0, 0), (0, 128 - D)))
    ip2 = jnp.pad(item_tab, ((0, 0), (0, 128 - D)))

    phys = hid.reshape(NW, BW * L)
    phys = jnp.pad(phys, ((0, 0), (0, NROWP * 128 - BW * L)))
    phys = phys.reshape(NW, NROWP, 128)

    histn = jnp.pad(hid, ((0, 0), (0, D - L))).reshape(NW, D, 128)

    out = _sc_call(up2, ip2, uid, iid, phys, histn)
    return out[:, :3 * D]
